# baseline (speedup 1.0000x reference)
import math

import jax
import jax.numpy as jnp
from jax.experimental import pallas as pl
from jax.experimental.pallas import tpu as pltpu

DIM = 64
BASE = 10000.0
LANES = 128


def _rope_kernel(cos_ref, sin_ref, uc_ref, us_ref):
    cols = cos_ref.shape[1]
    i = pl.program_id(0)

    @pl.when(i == 0)
    def _channel0():
        r = jax.lax.broadcasted_iota(jnp.int32, (32, LANES), 0)
        w = jnp.exp(r.astype(jnp.float32) * jnp.float32(-math.log(BASE) / 32.0))
        rc = jnp.cos(jnp.float32(LANES) * w)
        rs = jnp.sin(jnp.float32(LANES) * w)
        lane = jax.lax.broadcasted_iota(jnp.int32, (32, LANES), 1)
        phase = lane.astype(jnp.float32) * w
        c_chunks = [jnp.cos(phase)]
        s_chunks = [jnp.sin(phase)]
        for _ in range(cols // LANES - 1):
            c, s = c_chunks[-1], s_chunks[-1]
            c_chunks.append(c * rc - s * rs)
            s_chunks.append(s * rc + c * rs)
        u_c = jnp.concatenate(c_chunks, axis=1)
        u_s = jnp.concatenate(s_chunks, axis=1)
        uc_ref[...] = u_c
        us_ref[...] = u_s
        cos_ref[0:32, :] = u_c
        cos_ref[32:64, :] = u_c
        sin_ref[0:32, :] = u_s
        sin_ref[32:64, :] = u_s

    @pl.when(i == 1)
    def _channel1():
        u_c = uc_ref[...]
        u_s = us_ref[...]
        cos_ref[0:32, :] = u_c
        cos_ref[32:64, :] = u_c
        sin_ref[0:32, :] = u_s
        sin_ref[32:64, :] = u_s

    @pl.when(i == 2)
    def _channel2():
        cos_ref[...] = jnp.ones((64, cols), jnp.float32)
        sin_ref[...] = jnp.zeros((64, cols), jnp.float32)


def kernel(x):
    seq_len = x.shape[1]
    cos_p, sin_p = pl.pallas_call(
        _rope_kernel,
        grid=(3,),
        out_specs=[
            pl.BlockSpec((64, seq_len), lambda i: (i, 0)),
            pl.BlockSpec((64, seq_len), lambda i: (i, 0)),
        ],
        out_shape=[
            jax.ShapeDtypeStruct((192, seq_len), jnp.float32),
            jax.ShapeDtypeStruct((192, seq_len), jnp.float32),
        ],
        scratch_shapes=[
            pltpu.VMEM((32, seq_len), jnp.float32),
            pltpu.VMEM((32, seq_len), jnp.float32),
        ],
    )()
    shape = (1, seq_len, 3, 1, DIM)
    cos = cos_p.reshape(3, DIM, seq_len).transpose(2, 0, 1).reshape(shape)
    sin = sin_p.reshape(3, DIM, seq_len).transpose(2, 0, 1).reshape(shape)
    return cos, sin


# speedup vs baseline: 1.1292x; 1.1292x over previous
import math

import jax
import jax.numpy as jnp
from jax.experimental import pallas as pl
from jax.experimental.pallas import tpu as pltpu

DIM = 64
BASE = 10000.0
LANES = 128


def _rope_kernel(cos_hbm, sin_hbm, uc_ref, us_ref, one_ref, zero_ref, sems):
    cols = uc_ref.shape[1]
    r = jax.lax.broadcasted_iota(jnp.int32, (32, LANES), 0)
    w = jnp.exp(r.astype(jnp.float32) * jnp.float32(-math.log(BASE) / 32.0))
    rc = jnp.cos(jnp.float32(LANES) * w)
    rs = jnp.sin(jnp.float32(LANES) * w)
    lane = jax.lax.broadcasted_iota(jnp.int32, (32, LANES), 1)
    phase = lane.astype(jnp.float32) * w
    c_chunks = [jnp.cos(phase)]
    s_chunks = [jnp.sin(phase)]
    for _ in range(cols // LANES - 1):
        c, s = c_chunks[-1], s_chunks[-1]
        c_chunks.append(c * rc - s * rs)
        s_chunks.append(s * rc + c * rs)
    uc_ref[...] = jnp.concatenate(c_chunks, axis=1)
    us_ref[...] = jnp.concatenate(s_chunks, axis=1)
    one_ref[...] = jnp.ones((32, cols), jnp.float32)
    zero_ref[...] = jnp.zeros((32, cols), jnp.float32)

    copies = []
    for k in range(6):
        dst_c = cos_hbm.at[pl.ds(32 * k, 32), :]
        dst_s = sin_hbm.at[pl.ds(32 * k, 32), :]
        src_c = uc_ref if k < 4 else one_ref
        src_s = us_ref if k < 4 else zero_ref
        copies.append(pltpu.make_async_copy(src_c, dst_c, sems.at[2 * k]))
        copies.append(pltpu.make_async_copy(src_s, dst_s, sems.at[2 * k + 1]))
    for cp in copies:
        cp.start()
    for cp in copies:
        cp.wait()


def kernel(x):
    seq_len = x.shape[1]
    cos_p, sin_p = pl.pallas_call(
        _rope_kernel,
        grid=(1,),
        out_specs=[
            pl.BlockSpec(memory_space=pltpu.MemorySpace.HBM),
            pl.BlockSpec(memory_space=pltpu.MemorySpace.HBM),
        ],
        out_shape=[
            jax.ShapeDtypeStruct((192, seq_len), jnp.float32),
            jax.ShapeDtypeStruct((192, seq_len), jnp.float32),
        ],
        scratch_shapes=[
            pltpu.VMEM((32, seq_len), jnp.float32),
            pltpu.VMEM((32, seq_len), jnp.float32),
            pltpu.VMEM((32, seq_len), jnp.float32),
            pltpu.VMEM((32, seq_len), jnp.float32),
            pltpu.SemaphoreType.DMA((12,)),
        ],
    )()
    shape = (1, seq_len, 3, 1, DIM)
    cos = cos_p.reshape(3, DIM, seq_len).transpose(2, 0, 1).reshape(shape)
    sin = sin_p.reshape(3, DIM, seq_len).transpose(2, 0, 1).reshape(shape)
    return cos, sin


# speedup vs baseline: 1.1851x; 1.0496x over previous
import math

import jax
import jax.numpy as jnp
from jax.experimental import pallas as pl
from jax.experimental.pallas import tpu as pltpu

DIM = 64
BASE = 10000.0
LANES = 128


def _rope_kernel(cos_ref, sin_ref, uc_ref, us_ref):
    cols = cos_ref.shape[1]
    i = pl.program_id(0)

    @pl.when(i == 0)
    def _first_half():
        r = jax.lax.broadcasted_iota(jnp.int32, (32, LANES), 0)
        w = jnp.exp(r.astype(jnp.float32) * jnp.float32(-math.log(BASE) / 32.0))
        rc = jnp.cos(jnp.float32(LANES) * w)
        rs = jnp.sin(jnp.float32(LANES) * w)
        lane = jax.lax.broadcasted_iota(jnp.int32, (32, LANES), 1)
        phase = lane.astype(jnp.float32) * w
        c_chunks = [jnp.cos(phase)]
        s_chunks = [jnp.sin(phase)]
        for _ in range(cols // LANES - 1):
            c, s = c_chunks[-1], s_chunks[-1]
            c_chunks.append(c * rc - s * rs)
            s_chunks.append(s * rc + c * rs)
        u_c = jnp.concatenate(c_chunks, axis=1)
        u_s = jnp.concatenate(s_chunks, axis=1)
        uc_ref[...] = u_c
        us_ref[...] = u_s
        cos_ref[0:32, :] = u_c
        cos_ref[32:64, :] = u_c
        cos_ref[64:96, :] = u_c
        sin_ref[0:32, :] = u_s
        sin_ref[32:64, :] = u_s
        sin_ref[64:96, :] = u_s

    @pl.when(i == 1)
    def _second_half():
        cos_ref[0:32, :] = uc_ref[...]
        cos_ref[32:96, :] = jnp.ones((64, cols), jnp.float32)
        sin_ref[0:32, :] = us_ref[...]
        sin_ref[32:96, :] = jnp.zeros((64, cols), jnp.float32)


def kernel(x):
    seq_len = x.shape[1]
    cos_p, sin_p = pl.pallas_call(
        _rope_kernel,
        grid=(2,),
        out_specs=[
            pl.BlockSpec((96, seq_len), lambda i: (i, 0)),
            pl.BlockSpec((96, seq_len), lambda i: (i, 0)),
        ],
        out_shape=[
            jax.ShapeDtypeStruct((192, seq_len), jnp.float32),
            jax.ShapeDtypeStruct((192, seq_len), jnp.float32),
        ],
        scratch_shapes=[
            pltpu.VMEM((32, seq_len), jnp.float32),
            pltpu.VMEM((32, seq_len), jnp.float32),
        ],
    )()
    shape = (1, seq_len, 3, 1, DIM)
    cos = cos_p.reshape(3, DIM, seq_len).transpose(2, 0, 1).reshape(shape)
    sin = sin_p.reshape(3, DIM, seq_len).transpose(2, 0, 1).reshape(shape)
    return cos, sin
